# baseline (speedup 1.0000x reference)
import functools

import jax
import jax.numpy as jnp
from jax import lax
from jax.experimental import pallas as pl
from jax.experimental.pallas import tpu as pltpu
from jax.experimental.pallas import tpu_sc as plsc

B, S, D = 16, 4096, 256
KB = 6
NW = 32
ROWS_PER_W = KB * S // NW
CHUNK = 128
NCHUNK = ROWS_PER_W // CHUNK
L = 16

_mesh = plsc.VectorSubcoreMesh(core_axis_name="c", subcore_axis_name="s")


@functools.partial(
    pl.kernel,
    mesh=_mesh,
    out_type=jax.ShapeDtypeStruct((KB * S, D), jnp.float32),
    scratch_types=[
        pltpu.VMEM((NCHUNK, CHUNK), jnp.int32),
        pltpu.VMEM((CHUNK, D), jnp.float32),
        pltpu.VMEM((CHUNK, D), jnp.float32),
        pltpu.VMEM((CHUNK, D), jnp.float32),
        pltpu.SemaphoreType.DMA,
        pltpu.SemaphoreType.DMA,
        pltpu.SemaphoreType.DMA,
        pltpu.SemaphoreType.DMA,
        pltpu.SemaphoreType.DMA,
        pltpu.SemaphoreType.DMA,
    ],
)
def _zcurve_sc(x_hbm, idx_hbm, out_hbm, gidx_v,
               rows0_v, rows1_v, rows2_v,
               gsem0, gsem1, gsem2, ssem0, ssem1, ssem2):
    wid = lax.axis_index("s") * 2 + lax.axis_index("c")
    out_base = wid * ROWS_PER_W
    chunk0 = wid * NCHUNK

    pltpu.sync_copy(idx_hbm.at[wid], gidx_v)

    NBUF = 3
    bufs = (rows0_v, rows1_v, rows2_v)
    gsems = (gsem0, gsem1, gsem2)
    ssems = (ssem0, ssem1, ssem2)
    g_copies = [None] * NBUF
    s_copies = [None] * NBUF
    for c in range(NCHUNK):
        p = c % NBUF
        off = ((chunk0 + c) // (S // CHUNK)) * S
        for s in range(CHUNK // L):
            gidx_v[c, pl.ds(s * L, L)] = gidx_v[c, pl.ds(s * L, L)] + off
        if c >= NBUF:
            s_copies[p].wait()
        g_copies[p] = pltpu.async_copy(x_hbm.at[gidx_v.at[c]], bufs[p], gsems[p])
        if c > 0:
            q = (c - 1) % NBUF
            g_copies[q].wait()
            s_copies[q] = pltpu.async_copy(
                bufs[q], out_hbm.at[pl.ds(out_base + (c - 1) * CHUNK, CHUNK)],
                ssems[q],
            )
    q = (NCHUNK - 1) % NBUF
    g_copies[q].wait()
    s_copies[q] = pltpu.async_copy(
        bufs[q], out_hbm.at[pl.ds(out_base + (NCHUNK - 1) * CHUNK, CHUNK)],
        ssems[q],
    )
    for c in range(max(0, NCHUNK - NBUF), NCHUNK):
        s_copies[c % NBUF].wait()


def _tc_body(x_ref, o_ref):
    r3 = lax.broadcasted_iota(jnp.int32, (8, D), 0)
    mask = ((r3 >> 1) & 1) == 0
    for t in range(64):
        ti = ((t >> 5) & 1) * 4 + ((t >> 3) & 1) * 2 + ((t >> 1) & 1)
        tj = ((t >> 4) & 1) * 4 + ((t >> 2) & 1) * 2 + ((t >> 0) & 1)
        for g in range(8):
            i_h = 2 * ((g >> 2) & 1) + (g & 1)
            j_h = (g >> 1) & 1
            i_row = 8 * ti + 2 * i_h
            rowA = x_ref[0, i_row, pl.ds(8 * tj, 8), :]
            rowB = x_ref[0, i_row + 1, pl.ds(8 * tj, 8), :]
            jidx = 4 * j_h + 2 * ((r3 >> 2) & 1) + (r3 & 1)
            gA = jnp.take_along_axis(rowA, jidx, axis=0)
            gB = jnp.take_along_axis(rowB, jidx, axis=0)
            o_ref[0, pl.ds(64 * t + 8 * g, 8), :] = jnp.where(mask, gA, gB)


def _zcurve_tc(x4):
    nb = x4.shape[0]
    return pl.pallas_call(
        _tc_body,
        grid=(nb,),
        in_specs=[pl.BlockSpec((1, 64, 64, D), lambda b: (b, 0, 0, 0))],
        out_specs=pl.BlockSpec((1, S, D), lambda b: (b, 0, 0)),
        out_shape=jax.ShapeDtypeStruct((nb, S, D), jnp.float32),
    )(x4)


def kernel(x, forward_shuffle_idx):
    idx2d = forward_shuffle_idx.reshape(S // CHUNK, CHUNK)
    rows = (NCHUNK * jnp.arange(NW)[:, None] + jnp.arange(NCHUNK)[None, :]) % (
        S // CHUNK
    )
    idx3 = idx2d[rows]
    out_sc = _zcurve_sc(x[:KB].reshape(KB * S, D), idx3)
    out_tc = _zcurve_tc(x[KB:].reshape(B - KB, 64, 64, D))
    return jnp.concatenate([out_sc.reshape(KB, S, D), out_tc], axis=0)


# speedup vs baseline: 2.1393x; 2.1393x over previous
import functools

import jax
import jax.numpy as jnp
from jax import lax
from jax.experimental import pallas as pl
from jax.experimental.pallas import tpu as pltpu
from jax.experimental.pallas import tpu_sc as plsc

B, S, D = 16, 4096, 256
NW = 32
ROWS_PER_W = B * S // NW
CHUNK = 128
NCHUNK = ROWS_PER_W // CHUNK

_mesh = plsc.VectorSubcoreMesh(core_axis_name="c", subcore_axis_name="s")


@functools.partial(
    pl.kernel,
    mesh=_mesh,
    out_type=jax.ShapeDtypeStruct((B * S, D), jnp.float32),
    scratch_types=[
        pltpu.VMEM((NCHUNK, CHUNK), jnp.int32),
        pltpu.VMEM((CHUNK, D), jnp.float32),
        pltpu.VMEM((CHUNK, D), jnp.float32),
        pltpu.VMEM((CHUNK, D), jnp.float32),
        pltpu.SemaphoreType.DMA,
        pltpu.SemaphoreType.DMA,
        pltpu.SemaphoreType.DMA,
        pltpu.SemaphoreType.DMA,
        pltpu.SemaphoreType.DMA,
        pltpu.SemaphoreType.DMA,
    ],
)
def _zcurve_sc(x_hbm, idx_hbm, out_hbm, gidx_v,
               rows0_v, rows1_v, rows2_v,
               gsem0, gsem1, gsem2, ssem0, ssem1, ssem2):
    wid = lax.axis_index("s") * 2 + lax.axis_index("c")
    out_base = wid * ROWS_PER_W

    pltpu.sync_copy(idx_hbm.at[wid], gidx_v)

    NBUF = 3
    bufs = (rows0_v, rows1_v, rows2_v)
    gsems = (gsem0, gsem1, gsem2)
    ssems = (ssem0, ssem1, ssem2)
    g_copies = [None] * NBUF
    s_copies = [None] * NBUF
    for c in range(NCHUNK):
        p = c % NBUF
        if c >= NBUF:
            s_copies[p].wait()
        g_copies[p] = pltpu.async_copy(x_hbm.at[gidx_v.at[c]], bufs[p], gsems[p])
        if c > 0:
            q = (c - 1) % NBUF
            g_copies[q].wait()
            s_copies[q] = pltpu.async_copy(
                bufs[q], out_hbm.at[pl.ds(out_base + (c - 1) * CHUNK, CHUNK)],
                ssems[q],
            )
    q = (NCHUNK - 1) % NBUF
    g_copies[q].wait()
    s_copies[q] = pltpu.async_copy(
        bufs[q], out_hbm.at[pl.ds(out_base + (NCHUNK - 1) * CHUNK, CHUNK)],
        ssems[q],
    )
    for c in range(max(0, NCHUNK - NBUF), NCHUNK):
        s_copies[c % NBUF].wait()


def kernel(x, forward_shuffle_idx):
    idx3 = forward_shuffle_idx.reshape(2, NCHUNK, CHUNK)
    gidx = idx3[None, :, :, :] + (jnp.arange(B, dtype=jnp.int32) * S)[
        :, None, None, None
    ]
    gidx = gidx.reshape(NW, NCHUNK, CHUNK)
    out = _zcurve_sc(x.reshape(B * S, D), gidx)
    return out.reshape(B, S, D)


# speedup vs baseline: 2.1790x; 1.0186x over previous
import functools

import jax
import jax.numpy as jnp
from jax import lax
from jax.experimental import pallas as pl
from jax.experimental.pallas import tpu as pltpu
from jax.experimental.pallas import tpu_sc as plsc

B, S, D = 16, 4096, 256
NW = 32
ROWS_PER_W = B * S // NW
CHUNK = 128
NCHUNK = ROWS_PER_W // CHUNK

_mesh = plsc.VectorSubcoreMesh(core_axis_name="c", subcore_axis_name="s")


@functools.partial(
    pl.kernel,
    mesh=_mesh,
    out_type=jax.ShapeDtypeStruct((B * S, D), jnp.float32),
    scratch_types=[
        pltpu.VMEM((NCHUNK, CHUNK), jnp.int32),
        pltpu.VMEM((CHUNK, D), jnp.float32),
        pltpu.VMEM((CHUNK, D), jnp.float32),
        pltpu.VMEM((CHUNK, D), jnp.float32),
        pltpu.SemaphoreType.DMA,
        pltpu.SemaphoreType.DMA,
        pltpu.SemaphoreType.DMA,
        pltpu.SemaphoreType.DMA,
        pltpu.SemaphoreType.DMA,
        pltpu.SemaphoreType.DMA,
    ],
)
def _zcurve_sc(x_hbm, idx_hbm, out_hbm, gidx_v,
               rows0_v, rows1_v, rows2_v,
               gsem0, gsem1, gsem2, ssem0, ssem1, ssem2):
    wid = lax.axis_index("s") * 2 + lax.axis_index("c")
    out_base = wid * ROWS_PER_W

    pltpu.sync_copy(idx_hbm.at[wid], gidx_v)

    NBUF = 3
    bufs = (rows0_v, rows1_v, rows2_v)
    gsems = (gsem0, gsem1, gsem2)
    ssems = (ssem0, ssem1, ssem2)
    g_copies = [None] * NBUF
    s_copies = [None] * NBUF
    for c in range(NCHUNK):
        p = c % NBUF
        if c >= NBUF:
            s_copies[p].wait()
        g_copies[p] = pltpu.async_copy(x_hbm.at[gidx_v.at[c]], bufs[p], gsems[p])
        if c >= 2:
            q = (c - 2) % NBUF
            g_copies[q].wait()
            s_copies[q] = pltpu.async_copy(
                bufs[q], out_hbm.at[pl.ds(out_base + (c - 2) * CHUNK, CHUNK)],
                ssems[q],
            )
    for c in range(NCHUNK - 2, NCHUNK):
        q = c % NBUF
        g_copies[q].wait()
        s_copies[q] = pltpu.async_copy(
            bufs[q], out_hbm.at[pl.ds(out_base + c * CHUNK, CHUNK)],
            ssems[q],
        )
    for c in range(max(0, NCHUNK - NBUF), NCHUNK):
        s_copies[c % NBUF].wait()


def kernel(x, forward_shuffle_idx):
    idx3 = forward_shuffle_idx.reshape(2, NCHUNK, CHUNK)
    gidx = idx3[None, :, :, :] + (jnp.arange(B, dtype=jnp.int32) * S)[
        :, None, None, None
    ]
    gidx = gidx.reshape(NW, NCHUNK, CHUNK)
    out = _zcurve_sc(x.reshape(B * S, D), gidx)
    return out.reshape(B, S, D)
